# baseline (speedup 1.0000x reference)
import functools

import jax
import jax.numpy as jnp
from jax import lax
from jax.experimental import pallas as pl
from jax.experimental.pallas import tpu as pltpu
from jax.experimental.pallas import tpu_sc as plsc

VOCAB = 100000
EMBED_D = 64
BATCH = 1024
MAX_NORM = 1.0

_NUM_CORES = 2
_NUM_SUBCORES = 16
_NUM_WORKERS = _NUM_CORES * _NUM_SUBCORES
_B_PER_W = BATCH // _NUM_WORKERS

_VB = 2048


@functools.lru_cache(maxsize=None)
def _build_sc_gather():
    mesh = plsc.VectorSubcoreMesh(core_axis_name="c", subcore_axis_name="s")

    @functools.partial(
        pl.kernel,
        mesh=mesh,
        compiler_params=pltpu.CompilerParams(use_tc_tiling_on_sc=False),
        out_type=jax.ShapeDtypeStruct((BATCH, EMBED_D), jnp.float32),
        scratch_types=[
            pltpu.VMEM((_B_PER_W,), jnp.int32),
            pltpu.VMEM((_B_PER_W, EMBED_D), jnp.float32),
            pltpu.SemaphoreType.DMA,
        ],
    )
    def gather(idx_hbm, table_hbm, out_hbm, idx_v, rows_v, sem):
        wid = lax.axis_index("s") * _NUM_CORES + lax.axis_index("c")
        base = wid * _B_PER_W
        pltpu.sync_copy(idx_hbm.at[pl.ds(base, _B_PER_W)], idx_v)
        pltpu.async_copy(table_hbm.at[idx_v], rows_v, sem).wait()
        pltpu.sync_copy(rows_v, out_hbm.at[pl.ds(base, _B_PER_W)])

    return gather


def _proj_body(emb_ref, w_ref, b_ref, out_ref):
    emb = emb_ref[...]
    normsq = jnp.sum(emb * emb, axis=1, keepdims=True)
    norm = jnp.sqrt(normsq)
    scale = MAX_NORM / jnp.maximum(norm, MAX_NORM)
    emb = emb * scale
    out_ref[...] = (
        lax.dot_general(
            emb,
            w_ref[...],
            (((1,), (1,)), ((), ())),
            preferred_element_type=jnp.float32,
        )
        + b_ref[...]
    )


def kernel(indices, table, W, b):
    emb = _build_sc_gather()(indices.astype(jnp.int32), table)
    b2 = b.reshape(1, VOCAB)
    grid = pl.cdiv(VOCAB, _VB)
    out = pl.pallas_call(
        _proj_body,
        grid=(grid,),
        in_specs=[
            pl.BlockSpec((BATCH, EMBED_D), lambda j: (0, 0)),
            pl.BlockSpec((_VB, EMBED_D), lambda j: (j, 0)),
            pl.BlockSpec((1, _VB), lambda j: (0, j)),
        ],
        out_specs=pl.BlockSpec((BATCH, _VB), lambda j: (0, j)),
        out_shape=jax.ShapeDtypeStruct((BATCH, VOCAB), jnp.float32),
        compiler_params=pltpu.CompilerParams(
            dimension_semantics=("arbitrary",),
        ),
    )(emb, W, b2)
    return out


# speedup vs baseline: 1.0028x; 1.0028x over previous
import functools

import jax
import jax.numpy as jnp
from jax import lax
from jax.experimental import pallas as pl
from jax.experimental.pallas import tpu as pltpu
from jax.experimental.pallas import tpu_sc as plsc

VOCAB = 100000
EMBED_D = 64
BATCH = 1024
MAX_NORM = 1.0

_NUM_CORES = 2
_NUM_SUBCORES = 16
_NUM_WORKERS = _NUM_CORES * _NUM_SUBCORES
_B_PER_W = BATCH // _NUM_WORKERS

_VB = 2048
_NSTEPS = -(-VOCAB // _VB)
_TAIL = VOCAB - (_NSTEPS - 1) * _VB
_NBUF = 5


@functools.lru_cache(maxsize=None)
def _build_sc_gather():
    mesh = plsc.VectorSubcoreMesh(core_axis_name="c", subcore_axis_name="s")

    @functools.partial(
        pl.kernel,
        mesh=mesh,
        compiler_params=pltpu.CompilerParams(use_tc_tiling_on_sc=False),
        out_type=jax.ShapeDtypeStruct((BATCH, EMBED_D), jnp.float32),
        scratch_types=[
            pltpu.VMEM((_B_PER_W,), jnp.int32),
            pltpu.VMEM((_B_PER_W, EMBED_D), jnp.float32),
            pltpu.SemaphoreType.DMA,
        ],
    )
    def gather(idx_hbm, table_hbm, out_hbm, idx_v, rows_v, sem):
        wid = lax.axis_index("s") * _NUM_CORES + lax.axis_index("c")
        base = wid * _B_PER_W
        pltpu.sync_copy(idx_hbm.at[pl.ds(base, _B_PER_W)], idx_v)
        pltpu.async_copy(table_hbm.at[idx_v], rows_v, sem).wait()
        pltpu.sync_copy(rows_v, out_hbm.at[pl.ds(base, _B_PER_W)])

    return gather


def _proj_body(emb_ref, w_ref, b_ref, out_hbm, obuf, tailbuf, sems, tailsem):
    j = pl.program_id(0)
    slot = lax.rem(j, _NBUF)

    @pl.when(j >= _NBUF)
    def _():
        pltpu.make_async_copy(
            obuf.at[slot],
            out_hbm.at[:, pl.ds((j - _NBUF) * _VB, _VB)],
            sems.at[slot],
        ).wait()

    emb = emb_ref[...]
    normsq = jnp.sum(emb * emb, axis=1, keepdims=True)
    norm = jnp.sqrt(normsq)
    scale = MAX_NORM / jnp.maximum(norm, MAX_NORM)
    emb = emb * scale
    blk = (
        lax.dot_general(
            emb,
            w_ref[...],
            (((1,), (1,)), ((), ())),
            preferred_element_type=jnp.float32,
        )
        + b_ref[0]
    )

    @pl.when(j < _NSTEPS - 1)
    def _():
        obuf[slot] = blk
        pltpu.make_async_copy(
            obuf.at[slot],
            out_hbm.at[:, pl.ds(j * _VB, _VB)],
            sems.at[slot],
        ).start()

    @pl.when(j == _NSTEPS - 1)
    def _():
        tailbuf[...] = blk[:, :_TAIL]
        pltpu.make_async_copy(
            tailbuf,
            out_hbm.at[:, pl.ds((_NSTEPS - 1) * _VB, _TAIL)],
            tailsem,
        ).start()
        for k in range(_NBUF - 1):
            s = _NSTEPS - _NBUF + k
            pltpu.make_async_copy(
                obuf.at[s % _NBUF],
                out_hbm.at[:, pl.ds(s * _VB, _VB)],
                sems.at[s % _NBUF],
            ).wait()
        pltpu.make_async_copy(
            tailbuf,
            out_hbm.at[:, pl.ds((_NSTEPS - 1) * _VB, _TAIL)],
            tailsem,
        ).wait()


def kernel(indices, table, W, b):
    emb = _build_sc_gather()(indices.astype(jnp.int32), table)
    b2 = jnp.pad(b, (0, _NSTEPS * _VB - VOCAB)).reshape(_NSTEPS, 1, _VB)
    grid = _NSTEPS
    out = pl.pallas_call(
        _proj_body,
        grid=(grid,),
        in_specs=[
            pl.BlockSpec((BATCH, EMBED_D), lambda j: (0, 0)),
            pl.BlockSpec((_VB, EMBED_D), lambda j: (j, 0)),
            pl.BlockSpec((1, 1, _VB), lambda j: (j, 0, 0)),
        ],
        out_specs=pl.BlockSpec(memory_space=pl.ANY),
        out_shape=jax.ShapeDtypeStruct((BATCH, VOCAB), jnp.float32),
        scratch_shapes=[
            pltpu.VMEM((_NBUF, BATCH, _VB), jnp.float32),
            pltpu.VMEM((BATCH, _TAIL), jnp.float32),
            pltpu.SemaphoreType.DMA((_NBUF,)),
            pltpu.SemaphoreType.DMA,
        ],
        compiler_params=pltpu.CompilerParams(
            dimension_semantics=("arbitrary",),
        ),
    )(emb, W, b2)
    return out


# speedup vs baseline: 1.9176x; 1.9121x over previous
import functools

import jax
import jax.numpy as jnp
from jax import lax
from jax.experimental import pallas as pl
from jax.experimental.pallas import tpu as pltpu
from jax.experimental.pallas import tpu_sc as plsc

VOCAB = 100000
EMBED_D = 64
BATCH = 1024
MAX_NORM = 1.0

_NUM_CORES = 2
_NUM_SUBCORES = 16
_NUM_WORKERS = _NUM_CORES * _NUM_SUBCORES
_B_PER_W = BATCH // _NUM_WORKERS

_VB = 2000
_NSTEPS = VOCAB // _VB
_NBUF = 4


@functools.lru_cache(maxsize=None)
def _build_sc_gather():
    mesh = plsc.VectorSubcoreMesh(core_axis_name="c", subcore_axis_name="s")

    @functools.partial(
        pl.kernel,
        mesh=mesh,
        compiler_params=pltpu.CompilerParams(use_tc_tiling_on_sc=False),
        out_type=jax.ShapeDtypeStruct((BATCH, EMBED_D), jnp.float32),
        scratch_types=[
            pltpu.VMEM((_B_PER_W,), jnp.int32),
            pltpu.VMEM((_B_PER_W, EMBED_D), jnp.float32),
            pltpu.SemaphoreType.DMA,
        ],
    )
    def gather(idx_hbm, table_hbm, out_hbm, idx_v, rows_v, sem):
        wid = lax.axis_index("s") * _NUM_CORES + lax.axis_index("c")
        base = wid * _B_PER_W
        pltpu.sync_copy(idx_hbm.at[pl.ds(base, _B_PER_W)], idx_v)
        pltpu.async_copy(table_hbm.at[idx_v], rows_v, sem).wait()
        pltpu.sync_copy(rows_v, out_hbm.at[pl.ds(base, _B_PER_W)])

    return gather


def _proj_body(emb_ref, w_ref, b_ref, out_hbm, obuf, sems):
    j = pl.program_id(0)
    slot = lax.rem(j, _NBUF)

    @pl.when(j >= _NBUF)
    def _():
        pltpu.make_async_copy(
            obuf.at[slot],
            out_hbm.at[pl.ds((j - _NBUF) * _VB, _VB)],
            sems.at[slot],
        ).wait()

    emb = emb_ref[...]
    normsq = jnp.sum(emb * emb, axis=1, keepdims=True)
    norm = jnp.sqrt(normsq)
    scale = MAX_NORM / jnp.maximum(norm, MAX_NORM)
    emb = emb * scale
    obuf[slot] = (
        lax.dot_general(
            w_ref[...],
            emb,
            (((1,), (1,)), ((), ())),
            preferred_element_type=jnp.float32,
        )
        + b_ref[...]
    )
    pltpu.make_async_copy(
        obuf.at[slot],
        out_hbm.at[pl.ds(j * _VB, _VB)],
        sems.at[slot],
    ).start()

    @pl.when(j == _NSTEPS - 1)
    def _():
        for k in range(_NBUF):
            s = _NSTEPS - _NBUF + k
            pltpu.make_async_copy(
                obuf.at[s % _NBUF],
                out_hbm.at[pl.ds(s * _VB, _VB)],
                sems.at[s % _NBUF],
            ).wait()


def kernel(indices, table, W, b):
    emb = _build_sc_gather()(indices.astype(jnp.int32), table)
    b2 = b.reshape(VOCAB, 1)
    out_t = pl.pallas_call(
        _proj_body,
        grid=(_NSTEPS,),
        in_specs=[
            pl.BlockSpec((BATCH, EMBED_D), lambda j: (0, 0)),
            pl.BlockSpec((_VB, EMBED_D), lambda j: (j, 0)),
            pl.BlockSpec((_VB, 1), lambda j: (j, 0)),
        ],
        out_specs=pl.BlockSpec(memory_space=pl.ANY),
        out_shape=jax.ShapeDtypeStruct((VOCAB, BATCH), jnp.float32),
        scratch_shapes=[
            pltpu.VMEM((_NBUF, _VB, BATCH), jnp.float32),
            pltpu.SemaphoreType.DMA((_NBUF,)),
        ],
        compiler_params=pltpu.CompilerParams(
            dimension_semantics=("arbitrary",),
        ),
    )(emb, W, b2)
    return out_t.T


# speedup vs baseline: 2.8519x; 1.4873x over previous
import functools

import jax
import jax.numpy as jnp
from jax import lax
from jax.experimental import pallas as pl
from jax.experimental.pallas import tpu as pltpu
from jax.experimental.pallas import tpu_sc as plsc

VOCAB = 100000
EMBED_D = 64
BATCH = 1024
MAX_NORM = 1.0

_NUM_CORES = 2
_NUM_SUBCORES = 16
_NUM_WORKERS = _NUM_CORES * _NUM_SUBCORES
_B_PER_W = BATCH // _NUM_WORKERS

_VB = 2048
_NSTEPS = -(-VOCAB // _VB)
_TAIL = VOCAB - (_NSTEPS - 1) * _VB
_NBUF = 4


@functools.lru_cache(maxsize=None)
def _build_sc_gather():
    mesh = plsc.VectorSubcoreMesh(
        core_axis_name="c", subcore_axis_name="s", num_cores=_NUM_CORES
    )

    @functools.partial(
        pl.kernel,
        mesh=mesh,
        compiler_params=pltpu.CompilerParams(use_tc_tiling_on_sc=True),
        out_type=jax.ShapeDtypeStruct((BATCH, 2 * EMBED_D), jnp.float32),
        scratch_types=[
            pltpu.VMEM((_B_PER_W,), jnp.int32),
            pltpu.VMEM((_B_PER_W, 2 * EMBED_D), jnp.float32),
            pltpu.SemaphoreType.DMA,
        ],
    )
    def gather(idx2_hbm, table2_hbm, out_hbm, idx_v, rows_v, sem):
        wid = lax.axis_index("s") * _NUM_CORES + lax.axis_index("c")
        base = wid * _B_PER_W
        pltpu.sync_copy(idx2_hbm.at[pl.ds(base, _B_PER_W)], idx_v)
        pltpu.async_copy(table2_hbm.at[idx_v], rows_v, sem).wait()
        pltpu.sync_copy(rows_v, out_hbm.at[pl.ds(base, _B_PER_W)])

    return gather


def _proj_body(pairs_ref, par_ref, wt_ref, b_ref, out_hbm, obuf, sems):
    j = pl.program_id(0)
    slot = lax.rem(j, _NBUF)

    @pl.when(j >= _NBUF)
    def _():
        pltpu.make_async_copy(
            obuf.at[slot],
            out_hbm.at[pl.ds((j - _NBUF) * _VB, _VB)],
            sems.at[slot],
        ).wait()

    pairs = pairs_ref[...]
    par = par_ref[...]
    emb = pairs[:, :EMBED_D] * (1.0 - par) + pairs[:, EMBED_D:] * par
    normsq = jnp.sum(emb * emb, axis=1, keepdims=True)
    norm = jnp.sqrt(normsq)
    scale = MAX_NORM / jnp.maximum(norm, MAX_NORM)
    emb = emb * scale
    wt_aug = jnp.concatenate([wt_ref[...], b_ref[...]], axis=0)
    emb_aug = jnp.concatenate(
        [emb, jnp.ones((BATCH, 1), jnp.float32)], axis=1
    )
    obuf[slot] = lax.dot_general(
        wt_aug,
        emb_aug,
        (((0,), (1,)), ((), ())),
        preferred_element_type=jnp.float32,
    )

    @pl.when(j < _NSTEPS - 1)
    def _():
        pltpu.make_async_copy(
            obuf.at[slot],
            out_hbm.at[pl.ds(j * _VB, _VB)],
            sems.at[slot],
        ).start()

    @pl.when(j == _NSTEPS - 1)
    def _():
        pltpu.make_async_copy(
            obuf.at[slot, pl.ds(0, _TAIL)],
            out_hbm.at[pl.ds((_NSTEPS - 1) * _VB, _TAIL)],
            sems.at[slot],
        ).start()
        for k in range(_NBUF):
            s = _NSTEPS - _NBUF + k
            w = _TAIL if s == _NSTEPS - 1 else _VB
            pltpu.make_async_copy(
                obuf.at[s % _NBUF, pl.ds(0, w)],
                out_hbm.at[pl.ds(s * _VB, w)],
                sems.at[s % _NBUF],
            ).wait()


def kernel(indices, table, W, b):
    idx = indices.astype(jnp.int32)
    table2 = table.reshape(VOCAB // 2, 2 * EMBED_D)
    pairs = _build_sc_gather()(idx // 2, table2)
    par = (idx % 2).astype(jnp.float32).reshape(BATCH, 1)
    wt = W.T
    b2 = b.reshape(1, VOCAB)
    out_t = pl.pallas_call(
        _proj_body,
        grid=(_NSTEPS,),
        in_specs=[
            pl.BlockSpec((BATCH, 2 * EMBED_D), lambda j: (0, 0)),
            pl.BlockSpec((BATCH, 1), lambda j: (0, 0)),
            pl.BlockSpec((EMBED_D, _VB), lambda j: (0, j)),
            pl.BlockSpec((1, _VB), lambda j: (0, j)),
        ],
        out_specs=pl.BlockSpec(memory_space=pl.ANY),
        out_shape=jax.ShapeDtypeStruct((VOCAB, BATCH), jnp.float32),
        scratch_shapes=[
            pltpu.VMEM((_NBUF, _VB, BATCH), jnp.float32),
            pltpu.SemaphoreType.DMA((_NBUF,)),
        ],
        compiler_params=pltpu.CompilerParams(
            dimension_semantics=("arbitrary",),
        ),
    )(pairs, par, wt, b2)
    return out_t.T
